# baseline (device time: 466003 ns/iter reference)
import functools

import jax
import jax.numpy as jnp
from jax import lax
from jax.experimental import pallas as pl
from jax.experimental.pallas import tpu as pltpu

N_DEV = 16


def kernel(x, w_mat, scale_x, scale_w):
    m_total, k_blk = x.shape
    _, n = w_mat.shape
    m_blk = m_total // N_DEV

    x = x.astype(jnp.float8_e4m3fn)
    n_half = n // 2
    w_mat = jnp.stack(
        [w_mat[:, :n_half], w_mat[:, n_half:]]
    ).astype(jnp.float8_e5m2)

    def body(x_ref, w_ref, sx_ref, sw_ref, out_ref,
             wbuf, xtiles, wsend, wrecv, xsend, xrecv):
        me = lax.axis_index("i")
        right = lax.rem(me + 1, N_DEV)

        def mod(v):
            return lax.rem(v + N_DEV, N_DEV)

        barrier = pltpu.get_barrier_semaphore()
        for k in range(1, N_DEV):
            pl.semaphore_signal(barrier, inc=1, device_id=(mod(me + k),),
                                device_id_type=pl.DeviceIdType.MESH)
        pl.semaphore_wait(barrier, N_DEV - 1)

        x_sends = []
        for k in range(1, N_DEV):
            t = mod(me + k)
            rdma = pltpu.make_async_remote_copy(
                src_ref=x_ref.at[pl.ds(t * m_blk, m_blk), :],
                dst_ref=xtiles.at[me],
                send_sem=xsend.at[t],
                recv_sem=xrecv.at[me],
                device_id=(t,),
                device_id_type=pl.DeviceIdType.MESH,
            )
            rdma.start()
            x_sends.append(rdma)

        def wait_x_recv(origin):
            pltpu.make_async_remote_copy(
                src_ref=x_ref.at[pl.ds(0, m_blk), :], dst_ref=xtiles.at[origin],
                send_sem=xsend.at[origin], recv_sem=xrecv.at[origin],
                device_id=(right,), device_id_type=pl.DeviceIdType.MESH,
            ).wait_recv()

        def w_half_desc(src, origin, j):
            return pltpu.make_async_remote_copy(
                src_ref=src.at[j],
                dst_ref=wbuf.at[origin, j],
                send_sem=wsend.at[origin, j],
                recv_sem=wrecv.at[origin, j],
                device_id=(right,),
                device_id_type=pl.DeviceIdType.MESH,
            )

        def accum(xt_bf16, origin, j, first=False):
            wh = wbuf[origin, j].astype(jnp.bfloat16)
            prod = jnp.dot(xt_bf16, wh, preferred_element_type=jnp.float32)
            sl = pl.ds(j * n_half, n_half)
            if first:
                out_ref[:, sl] = prod
            else:
                out_ref[:, sl] += prod

        w_descs = []
        for j in range(2):
            d = w_half_desc(w_ref, me, j)
            d.start()
            w_descs.append(d)

        x_own = x_ref[pl.ds(me * m_blk, m_blk), :].astype(jnp.bfloat16)
        for j in range(2):
            out_ref[:, pl.ds(j * n_half, n_half)] = jnp.dot(
                x_own, w_ref[j].astype(jnp.bfloat16),
                preferred_element_type=jnp.float32)

        for h in range(N_DEV - 1):
            o = mod(me - h - 1)
            for j in range(2):
                w_half_desc(wbuf.at[o], o, j).wait_recv()
                if h < N_DEV - 2:
                    d = w_half_desc(wbuf.at[o], o, j)
                    d.start()
                    w_descs.append(d)
                if j == 0:
                    wait_x_recv(o)
                    xt = xtiles[o].astype(jnp.bfloat16)
                accum(xt, o, j)

        scale = sx_ref[0] * sw_ref[0]
        y = out_ref[:, :] * scale
        out_ref[:, :] = y / (1.0 + jnp.exp(-jnp.clip(y, -60.0, 60.0)))

        for rdma in x_sends + w_descs:
            rdma.wait_send()

        @functools.partial(pl.run_scoped, sem2=pltpu.SemaphoreType.REGULAR)
        def _(sem2):
            for k in range(1, N_DEV):
                pl.semaphore_signal(sem2, inc=1, device_id=(mod(me + k),),
                                    device_id_type=pl.DeviceIdType.MESH)
            pl.semaphore_wait(sem2, N_DEV - 1)

    return pl.pallas_call(
        body,
        out_shape=jax.ShapeDtypeStruct((m_blk, n), jnp.float32),
        in_specs=[
            pl.BlockSpec(memory_space=pltpu.VMEM),
            pl.BlockSpec(memory_space=pltpu.VMEM),
            pl.BlockSpec(memory_space=pltpu.SMEM),
            pl.BlockSpec(memory_space=pltpu.SMEM),
        ],
        out_specs=pl.BlockSpec(memory_space=pltpu.VMEM),
        scratch_shapes=[
            pltpu.VMEM((N_DEV, 2, k_blk, n // 2), w_mat.dtype),
            pltpu.VMEM((N_DEV, m_blk, k_blk), x.dtype),
            pltpu.SemaphoreType.DMA((N_DEV, 2)),
            pltpu.SemaphoreType.DMA((N_DEV, 2)),
            pltpu.SemaphoreType.DMA((N_DEV,)),
            pltpu.SemaphoreType.DMA((N_DEV,)),
        ],
        compiler_params=pltpu.CompilerParams(
            collective_id=0,
            vmem_limit_bytes=60 * 1024 * 1024,
        ),
    )(x, w_mat, scale_x, scale_w)


# device time: 407783 ns/iter; 1.1428x vs baseline; 1.1428x over previous
import functools

import jax
import jax.numpy as jnp
from jax import lax
from jax.experimental import pallas as pl
from jax.experimental.pallas import tpu as pltpu

N_DEV = 16


def kernel(x, w_mat, scale_x, scale_w):
    m_total, k_blk = x.shape
    _, n = w_mat.shape
    m_blk = m_total // N_DEV

    x = x.astype(jnp.float8_e4m3fn)
    w_mat = w_mat.astype(jnp.float8_e5m2)

    def body(x_ref, w_ref, sx_ref, sw_ref, out_ref,
             wbuf, xtiles, wsend, wrecv, xsend, xrecv):
        me = lax.axis_index("i")
        right = lax.rem(me + 1, N_DEV)

        def mod(v):
            return lax.rem(v + N_DEV, N_DEV)

        barrier = pltpu.get_barrier_semaphore()
        for k in range(1, N_DEV):
            pl.semaphore_signal(barrier, inc=1, device_id=(mod(me + k),),
                                device_id_type=pl.DeviceIdType.MESH)
        pl.semaphore_wait(barrier, N_DEV - 1)

        x_sends = []
        for k in range(1, N_DEV):
            t = mod(me + k)
            rdma = pltpu.make_async_remote_copy(
                src_ref=x_ref.at[pl.ds(t * m_blk, m_blk), :],
                dst_ref=xtiles.at[me],
                send_sem=xsend.at[t],
                recv_sem=xrecv.at[me],
                device_id=(t,),
                device_id_type=pl.DeviceIdType.MESH,
            )
            rdma.start()
            x_sends.append(rdma)

        def wait_x_recv(origin):
            pltpu.make_async_remote_copy(
                src_ref=x_ref.at[pl.ds(0, m_blk), :], dst_ref=xtiles.at[origin],
                send_sem=xsend.at[origin], recv_sem=xrecv.at[origin],
                device_id=(right,), device_id_type=pl.DeviceIdType.MESH,
            ).wait_recv()

        def w_desc(src, origin):
            return pltpu.make_async_remote_copy(
                src_ref=src,
                dst_ref=wbuf.at[origin],
                send_sem=wsend.at[origin],
                recv_sem=wrecv.at[origin],
                device_id=(right,),
                device_id_type=pl.DeviceIdType.MESH,
            )

        w_descs = []
        d0 = w_desc(w_ref, me)
        d0.start()
        w_descs.append(d0)

        x_own = x_ref[pl.ds(me * m_blk, m_blk), :].astype(jnp.bfloat16)
        out_ref[:, :] = jnp.dot(x_own, w_ref[:, :].astype(jnp.bfloat16),
                                preferred_element_type=jnp.float32)

        for h in range(N_DEV - 1):
            o = mod(me - h - 1)
            w_desc(wbuf.at[o], o).wait_recv()
            if h < N_DEV - 2:
                d = w_desc(wbuf.at[o], o)
                d.start()
                w_descs.append(d)
            wait_x_recv(o)
            xt = xtiles[o].astype(jnp.bfloat16)
            out_ref[:, :] += jnp.dot(xt, wbuf[o].astype(jnp.bfloat16),
                                     preferred_element_type=jnp.float32)

        scale = sx_ref[0] * sw_ref[0]
        y = out_ref[:, :] * scale
        out_ref[:, :] = y / (1.0 + jnp.exp(-jnp.clip(y, -60.0, 60.0)))

        for rdma in x_sends + w_descs:
            rdma.wait_send()

        @functools.partial(pl.run_scoped, sem2=pltpu.SemaphoreType.REGULAR)
        def _(sem2):
            for k in range(1, N_DEV):
                pl.semaphore_signal(sem2, inc=1, device_id=(mod(me + k),),
                                    device_id_type=pl.DeviceIdType.MESH)
            pl.semaphore_wait(sem2, N_DEV - 1)

    return pl.pallas_call(
        body,
        out_shape=jax.ShapeDtypeStruct((m_blk, n), jnp.float32),
        in_specs=[
            pl.BlockSpec(memory_space=pltpu.VMEM),
            pl.BlockSpec(memory_space=pltpu.VMEM),
            pl.BlockSpec(memory_space=pltpu.SMEM),
            pl.BlockSpec(memory_space=pltpu.SMEM),
        ],
        out_specs=pl.BlockSpec(memory_space=pltpu.VMEM),
        scratch_shapes=[
            pltpu.VMEM((N_DEV, k_blk, n), w_mat.dtype),
            pltpu.VMEM((N_DEV, m_blk, k_blk), x.dtype),
            pltpu.SemaphoreType.DMA((N_DEV,)),
            pltpu.SemaphoreType.DMA((N_DEV,)),
            pltpu.SemaphoreType.DMA((N_DEV,)),
            pltpu.SemaphoreType.DMA((N_DEV,)),
        ],
        compiler_params=pltpu.CompilerParams(
            collective_id=0,
            vmem_limit_bytes=60 * 1024 * 1024,
        ),
    )(x, w_mat, scale_x, scale_w)


# device time: 266387 ns/iter; 1.7493x vs baseline; 1.5308x over previous
import functools

import jax
import jax.numpy as jnp
from jax import lax
from jax.experimental import pallas as pl
from jax.experimental.pallas import tpu as pltpu

N_DEV = 16


def kernel(x, w_mat, scale_x, scale_w):
    m_total, k_blk = x.shape
    _, n = w_mat.shape
    m_blk = m_total // N_DEV

    x = x.astype(jnp.float8_e4m3fn)
    n_half = n // 2
    w_mat = jnp.stack(
        [w_mat[:, :n_half], w_mat[:, n_half:]]
    ).astype(jnp.float8_e5m2)

    def body(x_ref, w_ref, sx_ref, sw_ref, out_ref,
             wbuf, xtiles, wsend, wrecv, xsend, xrecv):
        me = lax.axis_index("i")

        def mod(v):
            return lax.rem(v + N_DEV, N_DEV)

        right1 = mod(me + 1)
        odd = lax.rem(me, 2) == 1
        right2 = jnp.where(odd, mod(me - 1), mod(me + 3))

        def pred2(v):
            return mod(v + jnp.where(lax.rem(v, 2) == 0, 1, 13))

        barrier = pltpu.get_barrier_semaphore()
        for k in range(1, N_DEV):
            pl.semaphore_signal(barrier, inc=1, device_id=(mod(me + k),),
                                device_id_type=pl.DeviceIdType.MESH)
        pl.semaphore_wait(barrier, N_DEV - 1)

        x_sends = []
        for k in range(1, N_DEV):
            t = mod(me + k)
            rdma = pltpu.make_async_remote_copy(
                src_ref=x_ref.at[pl.ds(t * m_blk, m_blk), :],
                dst_ref=xtiles.at[me],
                send_sem=xsend.at[t],
                recv_sem=xrecv.at[me],
                device_id=(t,),
                device_id_type=pl.DeviceIdType.MESH,
            )
            rdma.start()
            x_sends.append(rdma)

        def wait_x_recv(origin):
            pltpu.make_async_remote_copy(
                src_ref=x_ref.at[pl.ds(0, m_blk), :], dst_ref=xtiles.at[origin],
                send_sem=xsend.at[origin], recv_sem=xrecv.at[origin],
                device_id=(right1,), device_id_type=pl.DeviceIdType.MESH,
            ).wait_recv()

        def w_desc(src, origin, ring, target):
            return pltpu.make_async_remote_copy(
                src_ref=src,
                dst_ref=wbuf.at[origin, ring],
                send_sem=wsend.at[origin, ring],
                recv_sem=wrecv.at[origin, ring],
                device_id=(target,),
                device_id_type=pl.DeviceIdType.MESH,
            )

        def accum(origin, ring, first=False):
            xt = xtiles[origin].astype(jnp.bfloat16)
            prod = jnp.dot(xt, wbuf[origin, ring].astype(jnp.bfloat16),
                           preferred_element_type=jnp.float32)
            sl = pl.ds(ring * n_half, n_half)
            if first:
                out_ref[:, sl] = prod
            else:
                out_ref[:, sl] += prod

        w_descs = []
        for ring, tgt in ((0, right1), (1, right2)):
            d = w_desc(w_ref.at[ring], me, ring, tgt)
            d.start()
            w_descs.append(d)

        x_own = x_ref[pl.ds(me * m_blk, m_blk), :].astype(jnp.bfloat16)
        for ring in range(2):
            out_ref[:, pl.ds(ring * n_half, n_half)] = jnp.dot(
                x_own, w_ref[ring].astype(jnp.bfloat16),
                preferred_element_type=jnp.float32)

        for k in range(1, N_DEV):
            wait_x_recv(mod(me + k))

        ob = me
        for h in range(N_DEV - 1):
            oa = mod(me - h - 1)
            ob = pred2(ob)
            w_desc(wbuf.at[oa, 0], oa, 0, right1).wait_recv()
            if h < N_DEV - 2:
                d = w_desc(wbuf.at[oa, 0], oa, 0, right1)
                d.start()
                w_descs.append(d)
            w_desc(wbuf.at[ob, 1], ob, 1, right2).wait_recv()
            if h < N_DEV - 2:
                d = w_desc(wbuf.at[ob, 1], ob, 1, right2)
                d.start()
                w_descs.append(d)
            accum(oa, 0)
            accum(ob, 1)

        scale = sx_ref[0] * sw_ref[0]
        y = out_ref[:, :] * scale
        out_ref[:, :] = y / (1.0 + jnp.exp(-jnp.clip(y, -60.0, 60.0)))

        for rdma in x_sends + w_descs:
            rdma.wait_send()

        @functools.partial(pl.run_scoped, sem2=pltpu.SemaphoreType.REGULAR)
        def _(sem2):
            for k in range(1, N_DEV):
                pl.semaphore_signal(sem2, inc=1, device_id=(mod(me + k),),
                                    device_id_type=pl.DeviceIdType.MESH)
            pl.semaphore_wait(sem2, N_DEV - 1)

    return pl.pallas_call(
        body,
        out_shape=jax.ShapeDtypeStruct((m_blk, n), jnp.float32),
        in_specs=[
            pl.BlockSpec(memory_space=pltpu.VMEM),
            pl.BlockSpec(memory_space=pltpu.VMEM),
            pl.BlockSpec(memory_space=pltpu.SMEM),
            pl.BlockSpec(memory_space=pltpu.SMEM),
        ],
        out_specs=pl.BlockSpec(memory_space=pltpu.VMEM),
        scratch_shapes=[
            pltpu.VMEM((N_DEV, 2, k_blk, n // 2), w_mat.dtype),
            pltpu.VMEM((N_DEV, m_blk, k_blk), x.dtype),
            pltpu.SemaphoreType.DMA((N_DEV, 2)),
            pltpu.SemaphoreType.DMA((N_DEV, 2)),
            pltpu.SemaphoreType.DMA((N_DEV,)),
            pltpu.SemaphoreType.DMA((N_DEV,)),
        ],
        compiler_params=pltpu.CompilerParams(
            collective_id=0,
            vmem_limit_bytes=60 * 1024 * 1024,
        ),
    )(x, w_mat, scale_x, scale_w)


# device time: 264691 ns/iter; 1.7606x vs baseline; 1.0064x over previous
import functools

import jax
import jax.numpy as jnp
from jax import lax
from jax.experimental import pallas as pl
from jax.experimental.pallas import tpu as pltpu

N_DEV = 16


def kernel(x, w_mat, scale_x, scale_w):
    m_total, k_blk = x.shape
    _, n = w_mat.shape
    m_blk = m_total // N_DEV

    x = x.astype(jnp.float8_e4m3fn)
    n_half = n // 2
    w_mat = jnp.stack(
        [w_mat[:, :n_half], w_mat[:, n_half:]]
    ).astype(jnp.float8_e5m2)

    def body(x_ref, w_ref, sx_ref, sw_ref, out_ref,
             wbuf, xtiles, wsend, wrecv, xsend, xrecv):
        me = lax.axis_index("i")

        def mod(v):
            return lax.rem(v + N_DEV, N_DEV)

        right1 = mod(me + 1)
        odd = lax.rem(me, 2) == 1
        right2 = jnp.where(odd, mod(me - 1), mod(me + 3))

        def pred2(v):
            return mod(v + jnp.where(lax.rem(v, 2) == 0, 1, 13))

        barrier = pltpu.get_barrier_semaphore()
        for k in range(1, N_DEV):
            pl.semaphore_signal(barrier, inc=1, device_id=(mod(me + k),),
                                device_id_type=pl.DeviceIdType.MESH)
        pl.semaphore_wait(barrier, N_DEV - 1)

        x_sends = []
        for k in range(1, N_DEV):
            t = mod(me + k)
            rdma = pltpu.make_async_remote_copy(
                src_ref=x_ref.at[pl.ds(t * m_blk, m_blk), :],
                dst_ref=xtiles.at[me],
                send_sem=xsend.at[t],
                recv_sem=xrecv.at[me],
                device_id=(t,),
                device_id_type=pl.DeviceIdType.MESH,
            )
            rdma.start()
            x_sends.append(rdma)

        def wait_x_recv(origin):
            pltpu.make_async_remote_copy(
                src_ref=x_ref.at[pl.ds(0, m_blk), :], dst_ref=xtiles.at[origin],
                send_sem=xsend.at[origin], recv_sem=xrecv.at[origin],
                device_id=(right1,), device_id_type=pl.DeviceIdType.MESH,
            ).wait_recv()

        def w_desc(src, origin, ring, target):
            return pltpu.make_async_remote_copy(
                src_ref=src,
                dst_ref=wbuf.at[origin, ring],
                send_sem=wsend.at[origin, ring],
                recv_sem=wrecv.at[origin, ring],
                device_id=(target,),
                device_id_type=pl.DeviceIdType.MESH,
            )

        def accum(origin, ring, first=False):
            prod = jnp.dot(xtiles[origin], wbuf[origin, ring],
                           preferred_element_type=jnp.float32)
            sl = pl.ds(ring * n_half, n_half)
            if first:
                out_ref[:, sl] = prod
            else:
                out_ref[:, sl] += prod

        w_descs = []
        for ring, tgt in ((0, right1), (1, right2)):
            d = w_desc(w_ref.at[ring], me, ring, tgt)
            d.start()
            w_descs.append(d)

        x_own = x_ref[pl.ds(me * m_blk, m_blk), :]
        for ring in range(2):
            out_ref[:, pl.ds(ring * n_half, n_half)] = jnp.dot(
                x_own, w_ref[ring], preferred_element_type=jnp.float32)

        for k in range(1, N_DEV):
            wait_x_recv(mod(me + k))

        ob = me
        for h in range(N_DEV - 1):
            oa = mod(me - h - 1)
            ob = pred2(ob)
            w_desc(wbuf.at[oa, 0], oa, 0, right1).wait_recv()
            if h < N_DEV - 2:
                d = w_desc(wbuf.at[oa, 0], oa, 0, right1)
                d.start()
                w_descs.append(d)
            w_desc(wbuf.at[ob, 1], ob, 1, right2).wait_recv()
            if h < N_DEV - 2:
                d = w_desc(wbuf.at[ob, 1], ob, 1, right2)
                d.start()
                w_descs.append(d)
            accum(oa, 0)
            accum(ob, 1)

        scale = sx_ref[0] * sw_ref[0]
        y = out_ref[:, :] * scale
        out_ref[:, :] = y / (1.0 + jnp.exp(-jnp.clip(y, -60.0, 60.0)))

        for rdma in x_sends + w_descs:
            rdma.wait_send()

        @functools.partial(pl.run_scoped, sem2=pltpu.SemaphoreType.REGULAR)
        def _(sem2):
            for k in range(1, N_DEV):
                pl.semaphore_signal(sem2, inc=1, device_id=(mod(me + k),),
                                    device_id_type=pl.DeviceIdType.MESH)
            pl.semaphore_wait(sem2, N_DEV - 1)

    return pl.pallas_call(
        body,
        out_shape=jax.ShapeDtypeStruct((m_blk, n), jnp.float32),
        in_specs=[
            pl.BlockSpec(memory_space=pltpu.VMEM),
            pl.BlockSpec(memory_space=pltpu.VMEM),
            pl.BlockSpec(memory_space=pltpu.SMEM),
            pl.BlockSpec(memory_space=pltpu.SMEM),
        ],
        out_specs=pl.BlockSpec(memory_space=pltpu.VMEM),
        scratch_shapes=[
            pltpu.VMEM((N_DEV, 2, k_blk, n // 2), w_mat.dtype),
            pltpu.VMEM((N_DEV, m_blk, k_blk), x.dtype),
            pltpu.SemaphoreType.DMA((N_DEV, 2)),
            pltpu.SemaphoreType.DMA((N_DEV, 2)),
            pltpu.SemaphoreType.DMA((N_DEV,)),
            pltpu.SemaphoreType.DMA((N_DEV,)),
        ],
        compiler_params=pltpu.CompilerParams(
            collective_id=0,
            vmem_limit_bytes=60 * 1024 * 1024,
        ),
    )(x, w_mat, scale_x, scale_w)


# device time: 255073 ns/iter; 1.8269x vs baseline; 1.0377x over previous
import functools

import jax
import jax.numpy as jnp
from jax import lax
from jax.experimental import pallas as pl
from jax.experimental.pallas import tpu as pltpu

N_DEV = 16


def kernel(x, w_mat, scale_x, scale_w):
    m_total, k_blk = x.shape
    _, n = w_mat.shape
    m_blk = m_total // N_DEV

    x = x.astype(jnp.float8_e4m3fn)
    n_half = n // 2
    w_mat = jnp.stack(
        [w_mat[:, :n_half], w_mat[:, n_half:]]
    ).astype(jnp.float8_e5m2)

    def body(x_ref, w_ref, sx_ref, sw_ref, out_ref,
             wbuf, xtiles, wsend, wrecv, xsend, xrecv):
        me = lax.axis_index("i")

        def mod(v):
            return lax.rem(v + N_DEV, N_DEV)

        right1 = mod(me + 1)
        odd = lax.rem(me, 2) == 1
        right2 = jnp.where(odd, mod(me - 1), mod(me + 3))

        def pred2(v):
            return mod(v + jnp.where(lax.rem(v, 2) == 0, 1, 13))

        barrier = pltpu.get_barrier_semaphore()
        for k in range(1, N_DEV):
            pl.semaphore_signal(barrier, inc=1, device_id=(mod(me + k),),
                                device_id_type=pl.DeviceIdType.MESH)
        pl.semaphore_wait(barrier, N_DEV - 1)

        x_sends = []
        for k in range(1, N_DEV):
            t = mod(me + k)
            for ring in range(2):
                rdma = pltpu.make_async_remote_copy(
                    src_ref=x_ref.at[pl.ds(t * m_blk, m_blk), :],
                    dst_ref=xtiles.at[ring, me],
                    send_sem=xsend.at[t, ring],
                    recv_sem=xrecv.at[me, ring],
                    device_id=(t,),
                    device_id_type=pl.DeviceIdType.MESH,
                )
                rdma.start()
                x_sends.append(rdma)

        def wait_x_recv(origin, ring):
            pltpu.make_async_remote_copy(
                src_ref=x_ref.at[pl.ds(0, m_blk), :],
                dst_ref=xtiles.at[ring, origin],
                send_sem=xsend.at[origin, ring],
                recv_sem=xrecv.at[origin, ring],
                device_id=(right1,), device_id_type=pl.DeviceIdType.MESH,
            ).wait_recv()

        def w_desc(src, origin, ring, target):
            return pltpu.make_async_remote_copy(
                src_ref=src,
                dst_ref=wbuf.at[origin, ring],
                send_sem=wsend.at[origin, ring],
                recv_sem=wrecv.at[origin, ring],
                device_id=(target,),
                device_id_type=pl.DeviceIdType.MESH,
            )

        def accum(origin, ring):
            xt = xtiles[ring, origin].astype(jnp.bfloat16)
            prod = jnp.dot(xt, wbuf[origin, ring].astype(jnp.bfloat16),
                           preferred_element_type=jnp.float32)
            out_ref[:, pl.ds(ring * n_half, n_half)] += prod

        w_descs = []
        for ring, tgt in ((0, right1), (1, right2)):
            d = w_desc(w_ref.at[ring], me, ring, tgt)
            d.start()
            w_descs.append(d)

        x_own = x_ref[pl.ds(me * m_blk, m_blk), :].astype(jnp.bfloat16)
        for ring in range(2):
            out_ref[:, pl.ds(ring * n_half, n_half)] = jnp.dot(
                x_own, w_ref[ring].astype(jnp.bfloat16),
                preferred_element_type=jnp.float32)

        ob = me
        for h in range(N_DEV - 1):
            oa = mod(me - h - 1)
            ob = pred2(ob)
            w_desc(wbuf.at[oa, 0], oa, 0, right1).wait_recv()
            if h < N_DEV - 2:
                d = w_desc(wbuf.at[oa, 0], oa, 0, right1)
                d.start()
                w_descs.append(d)
            w_desc(wbuf.at[ob, 1], ob, 1, right2).wait_recv()
            if h < N_DEV - 2:
                d = w_desc(wbuf.at[ob, 1], ob, 1, right2)
                d.start()
                w_descs.append(d)
            wait_x_recv(oa, 0)
            accum(oa, 0)
            wait_x_recv(ob, 1)
            accum(ob, 1)

        scale = sx_ref[0] * sw_ref[0]
        y = out_ref[:, :] * scale
        out_ref[:, :] = y / (1.0 + jnp.exp(-jnp.clip(y, -60.0, 60.0)))

        for rdma in x_sends + w_descs:
            rdma.wait_send()

        @functools.partial(pl.run_scoped, sem2=pltpu.SemaphoreType.REGULAR)
        def _(sem2):
            for k in range(1, N_DEV):
                pl.semaphore_signal(sem2, inc=1, device_id=(mod(me + k),),
                                    device_id_type=pl.DeviceIdType.MESH)
            pl.semaphore_wait(sem2, N_DEV - 1)

    return pl.pallas_call(
        body,
        out_shape=jax.ShapeDtypeStruct((m_blk, n), jnp.float32),
        in_specs=[
            pl.BlockSpec(memory_space=pltpu.VMEM),
            pl.BlockSpec(memory_space=pltpu.VMEM),
            pl.BlockSpec(memory_space=pltpu.SMEM),
            pl.BlockSpec(memory_space=pltpu.SMEM),
        ],
        out_specs=pl.BlockSpec(memory_space=pltpu.VMEM),
        scratch_shapes=[
            pltpu.VMEM((N_DEV, 2, k_blk, n // 2), w_mat.dtype),
            pltpu.VMEM((2, N_DEV, m_blk, k_blk), x.dtype),
            pltpu.SemaphoreType.DMA((N_DEV, 2)),
            pltpu.SemaphoreType.DMA((N_DEV, 2)),
            pltpu.SemaphoreType.DMA((N_DEV, 2)),
            pltpu.SemaphoreType.DMA((N_DEV, 2)),
        ],
        compiler_params=pltpu.CompilerParams(
            collective_id=0,
            vmem_limit_bytes=60 * 1024 * 1024,
        ),
    )(x, w_mat, scale_x, scale_w)
